# baseline (speedup 1.0000x reference)
import functools

import jax
import jax.numpy as jnp
from jax import lax
from jax.experimental import pallas as pl
from jax.experimental.pallas import tpu as pltpu
from jax.experimental.pallas import tpu_sc as plsc

_INPUT_DIM = 4096
_NUM_EXPERTS = 64
_TOP_K = 8
_ALPHA = 10.0
_N_TOKENS = 8192

_NW = 32
_TPW = _N_TOKENS // _NW
_GROUPS = _TPW // 16


def _mm_body(x_ref, w_ref, b_ref, o_ref):
    lg = jax.lax.dot_general(
        x_ref[...], w_ref[...],
        dimension_numbers=(((1,), (1,)), ((), ())),
        preferred_element_type=jnp.float32) + b_ref[...]
    o_ref[...] = lg.T


def _sc_gate_body(lg_hbm, out_hbm, lg_v, out_v):
    wid = lax.axis_index("s") * 2 + lax.axis_index("c")
    base = wid * _TPW
    pltpu.sync_copy(lg_hbm.at[:, pl.ds(base, _TPW)], lg_v)

    lanes = lax.iota(jnp.int32, 16)
    neg_inf = jnp.float32(-jnp.inf)
    kf = jnp.full((16,), float(_TOP_K), jnp.float32)

    def group(g, _):
        col = g * 16
        v = [lg_v[e, pl.ds(col, 16)] for e in range(_NUM_EXPERTS)]

        t = list(v)
        removed = jnp.zeros((16,), jnp.float32)
        kth = jnp.full((16,), neg_inf)
        m0 = None
        for r in range(_TOP_K):
            m = t[0]
            for e in range(1, _NUM_EXPERTS):
                m = jnp.maximum(m, t[e])
            if r == 0:
                m0 = m
            cnt = jnp.zeros((16,), jnp.float32)
            for e in range(_NUM_EXPERTS):
                cnt = cnt + jnp.where(t[e] == m, 1.0, 0.0)
            total = removed + cnt
            hit = (jnp.where(removed < kf, 1.0, 0.0)
                   * jnp.where(total >= kf, 1.0, 0.0))
            kth = jnp.where(hit > 0.5, m, kth)
            removed = total
            if r < _TOP_K - 1:
                for e in range(_NUM_EXPERTS):
                    t[e] = jnp.where(t[e] == m, neg_inf, t[e])

        ex = [jnp.exp(v[e] - m0) for e in range(_NUM_EXPERTS)]
        s = ex[0]
        for e in range(1, _NUM_EXPERTS):
            s = s + ex[e]
        inv = 1.0 / s
        m1 = _ALPHA * (jnp.exp(inv) - 1.0)
        cm = jnp.exp(-m1)
        e1 = []
        for e in range(_NUM_EXPERTS):
            sm = ex[e] * inv
            p = 1.0 + sm
            p2 = p * p
            p4 = p2 * p2
            p8 = p4 * p4
            p10 = p8 * p2
            lo = p10 * cm
            hi = jnp.exp(_ALPHA * jnp.exp(sm) - (_ALPHA + m1))
            e1.append(jnp.where(v[e] < kth, lo, hi))
        s1 = e1[0]
        for e in range(1, _NUM_EXPERTS):
            s1 = s1 + e1[e]
        inv1 = 1.0 / s1
        for e in range(_NUM_EXPERTS):
            out_v[e, pl.ds(col, 16)] = e1[e] * inv1
        return _

    lax.fori_loop(0, _GROUPS, group, 0)
    pltpu.sync_copy(out_v, out_hbm.at[:, pl.ds(base, _TPW)])


@functools.partial(
    pl.kernel,
    mesh=plsc.VectorSubcoreMesh(core_axis_name="c", subcore_axis_name="s"),
    out_type=jax.ShapeDtypeStruct((_NUM_EXPERTS, _N_TOKENS), jnp.float32),
    scratch_types=[
        pltpu.VMEM((_NUM_EXPERTS, _TPW), jnp.float32),
        pltpu.VMEM((_NUM_EXPERTS, _TPW), jnp.float32),
    ],
)
def _sc_gate(lg_hbm, out_hbm, lg_v, out_v):
    _sc_gate_body(lg_hbm, out_hbm, lg_v, out_v)


@jax.jit
def kernel(x, W_gate, b_gate):
    b2 = b_gate.reshape(1, _NUM_EXPERTS)
    tm = 1024
    logits = pl.pallas_call(
        _mm_body,
        grid=(_N_TOKENS // tm,),
        in_specs=[
            pl.BlockSpec((tm, _INPUT_DIM), lambda i: (i, 0)),
            pl.BlockSpec((_NUM_EXPERTS, _INPUT_DIM), lambda i: (0, 0)),
            pl.BlockSpec((1, _NUM_EXPERTS), lambda i: (0, 0)),
        ],
        out_specs=pl.BlockSpec((_NUM_EXPERTS, tm), lambda i: (0, i)),
        out_shape=jax.ShapeDtypeStruct((_NUM_EXPERTS, _N_TOKENS),
                                       jnp.float32),
    )(x, W_gate, b2)
    gates_t = _sc_gate(logits)
    return gates_t.T


# speedup vs baseline: 2.1133x; 2.1133x over previous
import functools

import jax
import jax.numpy as jnp
from jax.experimental import pallas as pl

_INPUT_DIM = 4096
_NUM_EXPERTS = 64
_TOP_K = 8
_ALPHA = 10.0
_N_TOKENS = 8192


def _gate_body(x_ref, w_ref, b_ref, o_ref):
    logits = jax.lax.dot_general(
        x_ref[...], w_ref[...],
        dimension_numbers=(((1,), (1,)), ((), ())),
        preferred_element_type=jnp.float32)
    lt = logits.T + b_ref[...]
    neg_inf = jnp.float32(-jnp.inf)

    t = lt
    removed = jnp.zeros(lt.shape[1:], jnp.float32)[None, :]
    kth = jnp.full_like(removed, neg_inf)
    m0 = None
    for step in range(_TOP_K):
        m = jnp.max(t, axis=0, keepdims=True)
        if step == 0:
            m0 = m
        eq = t == m
        cnt = jnp.sum(jnp.where(eq, 1.0, 0.0), axis=0, keepdims=True)
        total = removed + cnt
        hit = jnp.logical_and(removed < float(_TOP_K),
                              total >= float(_TOP_K))
        kth = jnp.where(hit, m, kth)
        removed = total
        if step < _TOP_K - 1:
            t = jnp.where(eq, neg_inf, t)

    mask = lt < kth

    e0 = jnp.exp(lt - m0)
    inv_s = 1.0 / jnp.sum(e0, axis=0, keepdims=True)
    sm = e0 * inv_s

    m1 = _ALPHA * (jnp.exp(inv_s) - 1.0)
    p = 1.0 + sm
    p2 = p * p
    p4 = p2 * p2
    p10 = p4 * p4 * p2
    e1 = jnp.where(mask,
                   p10 * jnp.exp(-m1),
                   jnp.exp(_ALPHA * jnp.exp(sm) - (_ALPHA + m1)))
    gt = e1 * (1.0 / jnp.sum(e1, axis=0, keepdims=True))
    o_ref[...] = gt.T


@jax.jit
def kernel(x, W_gate, b_gate):
    b2 = b_gate.reshape(_NUM_EXPERTS, 1)
    tm = 1024
    grid = (_N_TOKENS // tm,)
    return pl.pallas_call(
        _gate_body,
        grid=grid,
        in_specs=[
            pl.BlockSpec((tm, _INPUT_DIM), lambda i: (i, 0)),
            pl.BlockSpec((_NUM_EXPERTS, _INPUT_DIM), lambda i: (0, 0)),
            pl.BlockSpec((_NUM_EXPERTS, 1), lambda i: (0, 0)),
        ],
        out_specs=pl.BlockSpec((tm, _NUM_EXPERTS), lambda i: (i, 0)),
        out_shape=jax.ShapeDtypeStruct((_N_TOKENS, _NUM_EXPERTS),
                                       jnp.float32),
    )(x, W_gate, b2)


# speedup vs baseline: 2.1232x; 1.0047x over previous
import jax
import jax.numpy as jnp
from jax.experimental import pallas as pl

_INPUT_DIM = 4096
_NUM_EXPERTS = 64
_TOP_K = 8
_ALPHA = 10.0
_N_TOKENS = 8192


def _gate_body(x_ref, w_ref, b_ref, o_ref):
    logits = jax.lax.dot_general(
        x_ref[...], w_ref[...],
        dimension_numbers=(((1,), (1,)), ((), ())),
        preferred_element_type=jnp.float32)
    lt = logits.T + b_ref[...]
    neg_inf = jnp.float32(-jnp.inf)

    t = lt
    removed = jnp.zeros(lt.shape[1:], jnp.float32)[None, :]
    kth = jnp.full_like(removed, neg_inf)
    m0 = None
    for step in range(_TOP_K):
        m = jnp.max(t, axis=0, keepdims=True)
        if step == 0:
            m0 = m
        eq = t == m
        cnt = jnp.sum(jnp.where(eq, 1.0, 0.0), axis=0, keepdims=True)
        total = removed + cnt
        hit = jnp.logical_and(removed < float(_TOP_K),
                              total >= float(_TOP_K))
        kth = jnp.where(hit, m, kth)
        removed = total
        if step < _TOP_K - 1:
            t = jnp.where(eq, neg_inf, t)

    mask = lt < kth

    e0 = jnp.exp(lt - m0)
    inv_s = 1.0 / jnp.sum(e0, axis=0, keepdims=True)
    sm = e0 * inv_s

    m1 = _ALPHA * (jnp.exp(inv_s) - 1.0)
    p = 1.0 + sm
    p2 = p * p
    p4 = p2 * p2
    p10 = p4 * p4 * p2
    e1 = jnp.where(mask,
                   p10 * jnp.exp(-m1),
                   jnp.exp(_ALPHA * jnp.exp(sm) - (_ALPHA + m1)))
    gt = e1 * (1.0 / jnp.sum(e1, axis=0, keepdims=True))
    o_ref[...] = gt.T


@jax.jit
def kernel(x, W_gate, b_gate):
    b2 = b_gate.reshape(_NUM_EXPERTS, 1)
    tm = 1024
    grid = (_N_TOKENS // tm,)
    return pl.pallas_call(
        _gate_body,
        grid=grid,
        in_specs=[
            pl.BlockSpec((tm, _INPUT_DIM), lambda i: (i, 0)),
            pl.BlockSpec((_NUM_EXPERTS, _INPUT_DIM), lambda i: (0, 0)),
            pl.BlockSpec((_NUM_EXPERTS, 1), lambda i: (0, 0)),
        ],
        out_specs=pl.BlockSpec((tm, _NUM_EXPERTS), lambda i: (i, 0)),
        out_shape=jax.ShapeDtypeStruct((_N_TOKENS, _NUM_EXPERTS),
                                       jnp.float32),
    )(x, W_gate, b2)
